# baseline (speedup 1.0000x reference)
import functools

import jax
import jax.numpy as jnp
from jax import lax
from jax.experimental import pallas as pl
from jax.experimental.pallas import tpu as pltpu
from jax.experimental.pallas import tpu_sc as plsc

BATCH = 16384
DIM = 128
LANES = 16
NUM_CORES = 2
NUM_SUBCORES = 16
NUM_WORKERS = NUM_CORES * NUM_SUBCORES
BPW = BATCH // NUM_WORKERS
CHUNK = 32
NCHUNK = BPW // CHUNK
CHUNKP = CHUNK + 1
KBLK = DIM // LANES


NSLOT = 12


def _body(uid_hbm, iid_hbm, uemb_hbm, iemb_hbm, out_hbm,
          uid_v, iid_v, u_bufs, v_bufs, tposed, out_v, sems, id_sems):
    wid = lax.axis_index("s") * NUM_CORES + lax.axis_index("c")
    base = wid * BPW
    cu_ids = pltpu.async_copy(uid_hbm.at[pl.ds(base, BPW)], uid_v,
                              id_sems.at[0])
    cv_ids = pltpu.async_copy(iid_hbm.at[pl.ds(base, BPW)], iid_v,
                              id_sems.at[1])
    cu_ids.wait()
    cv_ids.wait()

    lane_iota = lax.iota(jnp.int32, LANES)
    scat_base = lane_iota * CHUNKP

    def start(c, slot):
        cu = pltpu.async_copy(
            uemb_hbm.at[uid_v.at[pl.ds(c * CHUNK, CHUNK)]], u_bufs.at[slot],
            sems.at[slot, 0])
        cv = pltpu.async_copy(
            iemb_hbm.at[iid_v.at[pl.ds(c * CHUNK, CHUNK)]], v_bufs.at[slot],
            sems.at[slot, 1])
        return cu, cv

    pending = [start(c, c % NSLOT) for c in range(min(NSLOT - 1, NCHUNK))]
    for c in range(NCHUNK):
        slot = c % NSLOT
        if c + NSLOT - 1 < NCHUNK:
            pending.append(start(c + NSLOT - 1, (c + NSLOT - 1) % NSLOT))
        cu, cv = pending.pop(0)
        cu.wait()
        cv.wait()
        u_rows = u_bufs.at[slot]
        v_rows = v_bufs.at[slot]

        @plsc.parallel_loop(0, CHUNK, unroll=1)
        def _(r):
            p = [u_rows[r, pl.ds(k * LANES, LANES)]
                 * v_rows[r, pl.ds(k * LANES, LANES)] for k in range(KBLK)]
            while len(p) > 1:
                p = [p[i] + p[i + 1] for i in range(0, len(p), 2)]
            plsc.store_scatter(tposed, [scat_base + r], p[0])

        for g in range(CHUNK // LANES):
            s = tposed[pl.ds(g * LANES, LANES)]
            for l in range(1, LANES):
                s = s + tposed[pl.ds(l * CHUNKP + g * LANES, LANES)]
            out_v[pl.ds(c * CHUNK + g * LANES, LANES)] = s

    pltpu.sync_copy(out_v, out_hbm.at[pl.ds(base, BPW)])


_tt = functools.partial(
    pl.kernel,
    out_type=jax.ShapeDtypeStruct((BATCH,), jnp.float32),
    mesh=plsc.VectorSubcoreMesh(core_axis_name="c", subcore_axis_name="s"),
    compiler_params=pltpu.CompilerParams(needs_layout_passes=False),
    scratch_types=[
        pltpu.VMEM((BPW,), jnp.int32),
        pltpu.VMEM((BPW,), jnp.int32),
        pltpu.VMEM((NSLOT, CHUNK, DIM), jnp.float32),
        pltpu.VMEM((NSLOT, CHUNK, DIM), jnp.float32),
        pltpu.VMEM((LANES * CHUNKP,), jnp.float32),
        pltpu.VMEM((BPW,), jnp.float32),
        pltpu.SemaphoreType.DMA((NSLOT, 2)),
        pltpu.SemaphoreType.DMA((2,)),
    ],
)(_body)


@jax.jit
def kernel(user_ids, item_ids, user_emb, item_emb):
    return _tt(user_ids.astype(jnp.int32), item_ids.astype(jnp.int32),
               user_emb, item_emb)


# speedup vs baseline: 1.0361x; 1.0361x over previous
import functools

import jax
import jax.numpy as jnp
from jax import lax
from jax.experimental import pallas as pl
from jax.experimental.pallas import tpu as pltpu
from jax.experimental.pallas import tpu_sc as plsc

BATCH = 16384
DIM = 128
LANES = 16
NUM_CORES = 2
NUM_SUBCORES = 16
NUM_WORKERS = NUM_CORES * NUM_SUBCORES
BPW = BATCH // NUM_WORKERS
CHUNK = 64
NCHUNK = BPW // CHUNK
CHUNKP = CHUNK + 1
KBLK = DIM // LANES


NSLOT = 7


def _body(uid_hbm, iid_hbm, uemb_hbm, iemb_hbm, out_hbm,
          uid_v, iid_v, u_bufs, v_bufs, tposed, out_v, sems, id_sems):
    wid = lax.axis_index("s") * NUM_CORES + lax.axis_index("c")
    base = wid * BPW
    cu_ids = pltpu.async_copy(uid_hbm.at[pl.ds(base, BPW)], uid_v,
                              id_sems.at[0])
    cv_ids = pltpu.async_copy(iid_hbm.at[pl.ds(base, BPW)], iid_v,
                              id_sems.at[1])
    cu_ids.wait()
    cv_ids.wait()

    lane_iota = lax.iota(jnp.int32, LANES)
    scat_base = lane_iota * CHUNKP

    def start(c, slot):
        cu = pltpu.async_copy(
            uemb_hbm.at[uid_v.at[pl.ds(c * CHUNK, CHUNK)]], u_bufs.at[slot],
            sems.at[slot, 0])
        cv = pltpu.async_copy(
            iemb_hbm.at[iid_v.at[pl.ds(c * CHUNK, CHUNK)]], v_bufs.at[slot],
            sems.at[slot, 1])
        return cu, cv

    pending = [start(c, c % NSLOT) for c in range(min(NSLOT - 1, NCHUNK))]
    for c in range(NCHUNK):
        slot = c % NSLOT
        if c + NSLOT - 1 < NCHUNK:
            pending.append(start(c + NSLOT - 1, (c + NSLOT - 1) % NSLOT))
        cu, cv = pending.pop(0)
        cu.wait()
        cv.wait()
        u_rows = u_bufs.at[slot]
        v_rows = v_bufs.at[slot]

        @plsc.parallel_loop(0, CHUNK, unroll=1)
        def _(r):
            p = [u_rows[r, pl.ds(k * LANES, LANES)]
                 * v_rows[r, pl.ds(k * LANES, LANES)] for k in range(KBLK)]
            while len(p) > 1:
                p = [p[i] + p[i + 1] for i in range(0, len(p), 2)]
            plsc.store_scatter(tposed, [scat_base + r], p[0])

        for g in range(CHUNK // LANES):
            s = tposed[pl.ds(g * LANES, LANES)]
            for l in range(1, LANES):
                s = s + tposed[pl.ds(l * CHUNKP + g * LANES, LANES)]
            out_v[pl.ds(c * CHUNK + g * LANES, LANES)] = s

    pltpu.sync_copy(out_v, out_hbm.at[pl.ds(base, BPW)])


_tt = functools.partial(
    pl.kernel,
    out_type=jax.ShapeDtypeStruct((BATCH,), jnp.float32),
    mesh=plsc.VectorSubcoreMesh(core_axis_name="c", subcore_axis_name="s"),
    compiler_params=pltpu.CompilerParams(needs_layout_passes=False),
    scratch_types=[
        pltpu.VMEM((BPW,), jnp.int32),
        pltpu.VMEM((BPW,), jnp.int32),
        pltpu.VMEM((NSLOT, CHUNK, DIM), jnp.float32),
        pltpu.VMEM((NSLOT, CHUNK, DIM), jnp.float32),
        pltpu.VMEM((LANES * CHUNKP,), jnp.float32),
        pltpu.VMEM((BPW,), jnp.float32),
        pltpu.SemaphoreType.DMA((NSLOT, 2)),
        pltpu.SemaphoreType.DMA((2,)),
    ],
)(_body)


@jax.jit
def kernel(user_ids, item_ids, user_emb, item_emb):
    return _tt(user_ids.astype(jnp.int32), item_ids.astype(jnp.int32),
               user_emb, item_emb)
